# baseline (speedup 1.0000x reference)
import functools

import jax
import jax.numpy as jnp
from jax import lax
from jax.experimental import pallas as pl
from jax.experimental.pallas import tpu as pltpu
from jax.experimental.pallas import tpu_sc as plsc

_N = 1000000
_NBATCH = 16384
_NANC = 64

_info = plsc.get_sparse_core_info()
_NC, _NS, _L = _info.num_cores, _info.num_subcores, _info.num_lanes
_NW = _NC * _NS
_ROWS_PER_W = _NBATCH // _NW
_CHUNK = _ROWS_PER_W * _NANC
_VPR = _NANC // _L


def _sc_slice_softmax(zs1d, xs1d):
    mesh = plsc.VectorSubcoreMesh(core_axis_name="c", subcore_axis_name="s")

    @functools.partial(
        pl.kernel,
        mesh=mesh,
        compiler_params=pltpu.CompilerParams(needs_layout_passes=False),
        out_type=jax.ShapeDtypeStruct((_NBATCH * _NANC,), jnp.float32),
        scratch_types=[
            pltpu.VMEM((_L,), jnp.int32),
            pltpu.VMEM((_CHUNK,), jnp.float32),
        ],
    )
    def k(zs_hbm, xs_hbm, out_hbm, idx_v, buf):
        wid = lax.axis_index("s") * _NC + lax.axis_index("c")
        pltpu.sync_copy(xs_hbm.at[pl.ds(0, _L)], idx_v)
        idxstart = idx_v[...][0]

        src_off = (idxstart + wid * _ROWS_PER_W) * _NANC
        pltpu.sync_copy(zs_hbm.at[pl.ds(src_off, _CHUNK)], buf)

        def row(r, carry):
            base = r * _NANC
            a = [buf[pl.ds(base + j * _L, _L)] for j in range(_VPR)]
            m = jnp.max(jnp.maximum(jnp.maximum(a[0], a[1]),
                                    jnp.maximum(a[2], a[3])))
            e = [jnp.exp(x - m) for x in a]
            s = jnp.sum(e[0] + e[1] + e[2] + e[3])
            rinv = 1.0 / jnp.broadcast_to(s, (_L,))
            for j in range(_VPR):
                buf[pl.ds(base + j * _L, _L)] = e[j] * rinv
            return carry

        lax.fori_loop(0, _ROWS_PER_W, row, 0)
        pltpu.sync_copy(buf, out_hbm.at[pl.ds(wid * _CHUNK, _CHUNK)])

    return k(zs1d, xs1d)


def kernel(zs, xs):
    out1d = _sc_slice_softmax(zs.reshape(-1), xs.reshape(-1))
    return out1d.reshape(_NBATCH, _NANC)
